# baseline (speedup 1.0000x reference)
import functools
import math

import jax
import jax.numpy as jnp
from jax import lax
from jax.experimental import pallas as pl
from jax.experimental.pallas import tpu as pltpu
from jax.experimental.pallas import tpu_sc as plsc

DIM = 32
SCALE = math.sqrt(float(DIM))

NC, NS = 2, 16
NW = NC * NS
NA, NP = 4096, 200
PG = NP // 8
PB = 4
NB = NP // PB
RB = PB * 128

VPAD = 1000064
TC_ = VPAD // 128
KMAX = 246

_mesh = plsc.VectorSubcoreMesh(core_axis_name="c", subcore_axis_name="s")

_params = pltpu.CompilerParams(
    use_tc_tiling_on_sc=False, needs_layout_passes=False
)


@functools.partial(
    pl.kernel,
    out_type=jax.ShapeDtypeStruct((VPAD, DIM), jnp.float32),
    mesh=_mesh,
    compiler_params=_params,
    scratch_types=[
        pltpu.VMEM((3, DIM // 8, 8, 128), jnp.float32),
        pltpu.VMEM((3, 128, DIM + 1), jnp.float32),
    ]
    + [pltpu.SemaphoreType.DMA] * 3
    + [pltpu.SemaphoreType.DMA] * 3,
)
def _detile_sc(tv_hbm, out_hbm, in_v, oc_v, *sems):
    gsems, osems = sems[:3], sems[3:]
    wid = lax.axis_index("s") * NC + lax.axis_index("c")

    l_idx = [lax.iota(jnp.int32, 16) + m * 16 for m in range(8)]
    d_sp = [jnp.zeros((16,), jnp.int32) + d for d in range(DIM)]

    def fire_stage(k, slot):
        pltpu.async_copy(
            tv_hbm.at[:, wid + k * NW], in_v.at[slot], gsems[slot]
        )

    def wait_stage(slot):
        pltpu.make_async_copy(
            tv_hbm.at[:, 0], in_v.at[slot], gsems[slot]
        ).wait()

    def fire_wb(k, slot):
        pltpu.async_copy(
            oc_v.at[slot].at[:, pl.ds(0, DIM)],
            out_hbm.at[pl.ds((wid + k * NW) * 128, 128)],
            osems[slot],
        )

    def wait_wb(slot):
        pltpu.make_async_copy(
            oc_v.at[slot].at[:, pl.ds(0, DIM)],
            out_hbm.at[pl.ds(0, 128)],
            osems[slot],
        ).wait()

    fire_stage(0, 0)

    @pl.when(wid + NW < TC_)
    def _():
        fire_stage(1, 1)

    def outer(g, carry):
        for b in range(3):
            k = g * 3 + b
            c = wid + k * NW

            @pl.when(c < TC_)
            def _():
                wait_stage(b)

                @pl.when(c + 2 * NW < TC_)
                def _():
                    fire_stage(k + 2, (b + 2) % 3)

                @pl.when(k >= 3)
                def _():
                    wait_wb(b)

                oc = oc_v.at[b]
                for dg in range(DIM // 8):
                    for sx in range(8):
                        row = in_v.at[b].at[dg, sx]
                        for m in range(8):
                            plsc.store_scatter(
                                oc,
                                [l_idx[m], d_sp[dg * 8 + sx]],
                                row[pl.ds(m * 16, 16)],
                            )
                fire_wb(k, b)
        return carry

    lax.fori_loop(0, KMAX // 3, outer, 0)

    for slot in range(3):
        wait_wb(slot)


@functools.partial(
    pl.kernel,
    out_type=jax.ShapeDtypeStruct((NP, DIM // 8, NW, 8, 128), jnp.float32),
    mesh=_mesh,
    compiler_params=_params,
    scratch_types=[
        pltpu.VMEM((PG, 8, 128), jnp.int32),
        pltpu.VMEM((2, RB, DIM), jnp.float32),
        pltpu.VMEM((2, PB, DIM // 8, 8, 129), jnp.float32),
    ]
    + [pltpu.SemaphoreType.DMA] * 2
    + [pltpu.SemaphoreType.DMA] * 2,
)
def _embed_sc(x_hbm, table_hbm, out_hbm, x_v, rows_v, t_v, *sems):
    gsems, osems = sems[:2], sems[2:]
    wid = lax.axis_index("s") * NC + lax.axis_index("c")

    pltpu.sync_copy(x_hbm.at[:, wid], x_v)

    d_iota = lax.iota(jnp.int32, 16)
    r_idx = [d_iota // 8 + 2 * h for h in range(2)]
    s_idx = lax.rem(d_iota, 8)

    def fire_gather(bi, slot):
        for j in range(PB):
            p = bi * PB + j
            pg = p // 8
            sx = lax.rem(p, 8)
            pltpu.async_copy(
                table_hbm.at[x_v.at[pg, sx]],
                rows_v.at[slot].at[pl.ds(j * 128, 128)],
                gsems[slot],
            )

    def wait_gather(slot):
        pltpu.make_async_copy(
            table_hbm.at[pl.ds(0, RB)], rows_v.at[slot], gsems[slot]
        ).wait()

    def fire_wb(bi, slot):
        pltpu.async_copy(
            t_v.at[slot].at[:, :, :, pl.ds(0, 128)],
            out_hbm.at[pl.ds(bi * PB, PB), :, wid],
            osems[slot],
        )

    def wait_wb(slot):
        pltpu.make_async_copy(
            t_v.at[slot].at[:, :, :, pl.ds(0, 128)],
            out_hbm.at[pl.ds(0, PB), :, 0],
            osems[slot],
        ).wait()

    fire_gather(0, 0)

    def outer(g, carry):
        for b in range(2):
            bi = g * 2 + b
            wait_gather(b)

            @pl.when(bi + 1 < NB)
            def _():
                fire_gather(bi + 1, 1 - b)

            @pl.when(bi >= 2)
            def _():
                wait_wb(b)

            rows = rows_v.at[b]
            for j in range(PB):
                tj = t_v.at[b].at[j]

                @plsc.parallel_loop(0, 128, 1, unroll=4)
                def _(l):
                    l_sp = jnp.zeros((16,), jnp.int32) + l
                    for h in range(2):
                        vals = rows[j * 128 + l, pl.ds(h * 16, 16)] * SCALE
                        plsc.store_scatter(
                            tj, [r_idx[h], s_idx, l_sp], vals
                        )

            fire_wb(bi, b)
        return carry

    lax.fori_loop(0, NB // 2, outer, 0)

    for slot in range(2):
        wait_wb(slot)


def kernel(x, table):
    xt = (
        x.astype(jnp.int32)
        .T.reshape(PG, 8, NW, 128)
        .transpose(0, 2, 1, 3)
    )
    tpad = jnp.pad(table, ((0, VPAD - table.shape[0]), (0, 0)))
    tv = tpad.T.reshape(DIM // 8, 8, TC_, 128).transpose(0, 2, 1, 3)
    trm = _detile_sc(tv)
    v = _embed_sc(xt, trm)
    return v.transpose(2, 4, 0, 1, 3).reshape(NA, NP, DIM)


# speedup vs baseline: 1.1032x; 1.1032x over previous
import functools
import math

import jax
import jax.numpy as jnp
from jax import lax
from jax.experimental import pallas as pl
from jax.experimental.pallas import tpu as pltpu
from jax.experimental.pallas import tpu_sc as plsc

DIM = 32
SCALE = math.sqrt(float(DIM))

NC, NS = 2, 16
NW = NC * NS
NA, NP = 4096, 200
PG = NP // 8
PB = 4
NB = NP // PB
RB = PB * 128

VPAD = 1000064
TC_ = VPAD // 128
KMAX = 246

_mesh = plsc.VectorSubcoreMesh(core_axis_name="c", subcore_axis_name="s")

_params = pltpu.CompilerParams(
    use_tc_tiling_on_sc=False, needs_layout_passes=False
)


@functools.partial(
    pl.kernel,
    out_type=jax.ShapeDtypeStruct((VPAD, DIM), jnp.float32),
    mesh=_mesh,
    compiler_params=_params,
    scratch_types=[
        pltpu.VMEM((2, DIM // 8, 8, 128), jnp.float32),
        pltpu.VMEM((2, 128, DIM + 1), jnp.float32),
    ]
    + [pltpu.SemaphoreType.DMA] * 2
    + [pltpu.SemaphoreType.DMA] * 2,
)
def _detile_sc(tv_hbm, out_hbm, in_v, oc_v, *sems):
    gsems, osems = sems[:2], sems[2:]
    wid = lax.axis_index("s") * NC + lax.axis_index("c")

    l_idx = [lax.iota(jnp.int32, 16) + m * 16 for m in range(8)]
    d_sp = [jnp.zeros((16,), jnp.int32) + d for d in range(DIM)]

    def fire_stage(k, slot):
        pltpu.async_copy(
            tv_hbm.at[:, wid + k * NW], in_v.at[slot], gsems[slot]
        )

    def wait_stage(slot):
        pltpu.make_async_copy(
            tv_hbm.at[:, 0], in_v.at[slot], gsems[slot]
        ).wait()

    def fire_wb(k, slot):
        pltpu.async_copy(
            oc_v.at[slot].at[:, pl.ds(0, DIM)],
            out_hbm.at[pl.ds((wid + k * NW) * 128, 128)],
            osems[slot],
        )

    def wait_wb(slot):
        pltpu.make_async_copy(
            oc_v.at[slot].at[:, pl.ds(0, DIM)],
            out_hbm.at[pl.ds(0, 128)],
            osems[slot],
        ).wait()

    fire_stage(0, 0)

    def outer(g, carry):
        for b in range(2):
            k = g * 2 + b
            c = wid + k * NW

            @pl.when(c < TC_)
            def _():
                wait_stage(b)

                @pl.when(c + NW < TC_)
                def _():
                    fire_stage(k + 1, 1 - b)

                @pl.when(k >= 2)
                def _():
                    wait_wb(b)

                oc = oc_v.at[b]
                for dg in range(DIM // 8):
                    for sx in range(8):
                        row = in_v.at[b].at[dg, sx]
                        for m in range(8):
                            plsc.store_scatter(
                                oc,
                                [l_idx[m], d_sp[dg * 8 + sx]],
                                row[pl.ds(m * 16, 16)],
                            )
                fire_wb(k, b)
        return carry

    lax.fori_loop(0, KMAX // 2, outer, 0)

    for slot in range(2):
        wait_wb(slot)


@functools.partial(
    pl.kernel,
    out_type=jax.ShapeDtypeStruct((NP, DIM // 8, NW, 8, 128), jnp.float32),
    mesh=_mesh,
    compiler_params=_params,
    scratch_types=[
        pltpu.VMEM((PG, 8, 128), jnp.int32),
        pltpu.VMEM((2, RB, DIM), jnp.float32),
        pltpu.VMEM((2, PB, DIM // 8, 8, 129), jnp.float32),
    ]
    + [pltpu.SemaphoreType.DMA] * 2
    + [pltpu.SemaphoreType.DMA] * 2,
)
def _embed_sc(x_hbm, table_hbm, out_hbm, x_v, rows_v, t_v, *sems):
    gsems, osems = sems[:2], sems[2:]
    wid = lax.axis_index("s") * NC + lax.axis_index("c")

    pltpu.sync_copy(x_hbm.at[:, wid], x_v)

    d_iota = lax.iota(jnp.int32, 16)
    r_idx = [d_iota // 8 + 2 * h for h in range(2)]
    s_idx = lax.rem(d_iota, 8)

    def fire_gather(bi, slot):
        for j in range(PB):
            p = bi * PB + j
            pg = p // 8
            sx = lax.rem(p, 8)
            pltpu.async_copy(
                table_hbm.at[x_v.at[pg, sx]],
                rows_v.at[slot].at[pl.ds(j * 128, 128)],
                gsems[slot],
            )

    def wait_gather(slot):
        pltpu.make_async_copy(
            table_hbm.at[pl.ds(0, RB)], rows_v.at[slot], gsems[slot]
        ).wait()

    def fire_wb(bi, slot):
        pltpu.async_copy(
            t_v.at[slot].at[:, :, :, pl.ds(0, 128)],
            out_hbm.at[pl.ds(bi * PB, PB), :, wid],
            osems[slot],
        )

    def wait_wb(slot):
        pltpu.make_async_copy(
            t_v.at[slot].at[:, :, :, pl.ds(0, 128)],
            out_hbm.at[pl.ds(0, PB), :, 0],
            osems[slot],
        ).wait()

    fire_gather(0, 0)

    def outer(g, carry):
        for b in range(2):
            bi = g * 2 + b
            wait_gather(b)

            @pl.when(bi + 1 < NB)
            def _():
                fire_gather(bi + 1, 1 - b)

            @pl.when(bi >= 2)
            def _():
                wait_wb(b)

            rows = rows_v.at[b]
            for j in range(PB):
                tj = t_v.at[b].at[j]

                @plsc.parallel_loop(0, 128, 1, unroll=4)
                def _(l):
                    l_sp = jnp.zeros((16,), jnp.int32) + l
                    for h in range(2):
                        vals = rows[j * 128 + l, pl.ds(h * 16, 16)] * SCALE
                        plsc.store_scatter(
                            tj, [r_idx[h], s_idx, l_sp], vals
                        )

            fire_wb(bi, b)
        return carry

    lax.fori_loop(0, NB // 2, outer, 0)

    for slot in range(2):
        wait_wb(slot)


def kernel(x, table):
    xt = (
        x.astype(jnp.int32)
        .T.reshape(PG, 8, NW, 128)
        .transpose(0, 2, 1, 3)
    )
    tpad = jnp.pad(table, ((0, VPAD - table.shape[0]), (0, 0)))
    tv = tpad.T.reshape(DIM // 8, 8, TC_, 128).transpose(0, 2, 1, 3)
    trm = _detile_sc(tv)
    v = _embed_sc(xt, trm)
    return v.transpose(2, 4, 0, 1, 3).reshape(NA, NP, DIM)


# speedup vs baseline: 1.4552x; 1.3191x over previous
import functools
import math

import jax
import jax.numpy as jnp
from jax import lax
from jax.experimental import pallas as pl
from jax.experimental.pallas import tpu as pltpu
from jax.experimental.pallas import tpu_sc as plsc

DIM = 32
SCALE = math.sqrt(float(DIM))

NC, NS = 2, 16
NW = NC * NS
NA, NP = 4096, 200
PG = NP // 8
PB = 4
NB = NP // PB
RB = PB * 128

VPAD = 1000064
TC_ = VPAD // 128
KMAX = 246

_mesh = plsc.VectorSubcoreMesh(core_axis_name="c", subcore_axis_name="s")

_params = pltpu.CompilerParams(
    use_tc_tiling_on_sc=False, needs_layout_passes=False
)


@functools.partial(
    pl.kernel,
    out_type=jax.ShapeDtypeStruct((VPAD, DIM), jnp.float32),
    mesh=_mesh,
    compiler_params=_params,
    scratch_types=[
        pltpu.VMEM((2, DIM // 8, 8, 128), jnp.float32),
        pltpu.VMEM((2, 128, DIM + 1), jnp.float32),
    ]
    + [pltpu.SemaphoreType.DMA] * 2
    + [pltpu.SemaphoreType.DMA] * 2,
)
def _detile_sc(tv_hbm, out_hbm, in_v, oc_v, *sems):
    gsems, osems = sems[:2], sems[2:]
    wid = lax.axis_index("s") * NC + lax.axis_index("c")

    l_idx = [lax.iota(jnp.int32, 16) + m * 16 for m in range(8)]

    def fire_stage(k, slot):
        pltpu.async_copy(
            tv_hbm.at[:, wid + k * NW], in_v.at[slot], gsems[slot]
        )

    def wait_stage(slot):
        pltpu.make_async_copy(
            tv_hbm.at[:, 0], in_v.at[slot], gsems[slot]
        ).wait()

    def fire_wb(k, slot):
        pltpu.async_copy(
            oc_v.at[slot].at[:, pl.ds(0, DIM)],
            out_hbm.at[pl.ds((wid + k * NW) * 128, 128)],
            osems[slot],
        )

    def wait_wb(slot):
        pltpu.make_async_copy(
            oc_v.at[slot].at[:, pl.ds(0, DIM)],
            out_hbm.at[pl.ds(0, 128)],
            osems[slot],
        ).wait()

    fire_stage(0, 0)

    def outer(g, carry):
        for b in range(2):
            k = g * 2 + b
            c = wid + k * NW

            @pl.when(c < TC_)
            def _():
                wait_stage(b)

                @pl.when(c + NW < TC_)
                def _():
                    fire_stage(k + 1, 1 - b)

                @pl.when(k >= 2)
                def _():
                    wait_wb(b)

                oc = oc_v.at[b]
                for dg in range(DIM // 8):
                    iv = in_v.at[b].at[dg]

                    @plsc.parallel_loop(0, 8, 1, unroll=2)
                    def _(sx):
                        d_sp = jnp.zeros((16,), jnp.int32) + (dg * 8 + sx)
                        for m in range(8):
                            plsc.store_scatter(
                                oc,
                                [l_idx[m], d_sp],
                                iv[sx, pl.ds(m * 16, 16)],
                            )
                fire_wb(k, b)
        return carry

    lax.fori_loop(0, KMAX // 2, outer, 0)

    for slot in range(2):
        wait_wb(slot)


@functools.partial(
    pl.kernel,
    out_type=jax.ShapeDtypeStruct((NP, DIM // 8, NW, 8, 128), jnp.float32),
    mesh=_mesh,
    compiler_params=_params,
    scratch_types=[
        pltpu.VMEM((PG, 8, 128), jnp.int32),
        pltpu.VMEM((2, RB, DIM), jnp.float32),
        pltpu.VMEM((2, PB, DIM // 8, 8, 129), jnp.float32),
    ]
    + [pltpu.SemaphoreType.DMA] * 2
    + [pltpu.SemaphoreType.DMA] * 2,
)
def _embed_sc(x_hbm, table_hbm, out_hbm, x_v, rows_v, t_v, *sems):
    gsems, osems = sems[:2], sems[2:]
    wid = lax.axis_index("s") * NC + lax.axis_index("c")

    pltpu.sync_copy(x_hbm.at[:, wid], x_v)

    d_iota = lax.iota(jnp.int32, 16)
    r_idx = [d_iota // 8 + 2 * h for h in range(2)]
    s_idx = lax.rem(d_iota, 8)

    def fire_gather(bi, slot):
        for j in range(PB):
            p = bi * PB + j
            pg = p // 8
            sx = lax.rem(p, 8)
            pltpu.async_copy(
                table_hbm.at[x_v.at[pg, sx]],
                rows_v.at[slot].at[pl.ds(j * 128, 128)],
                gsems[slot],
            )

    def wait_gather(slot):
        pltpu.make_async_copy(
            table_hbm.at[pl.ds(0, RB)], rows_v.at[slot], gsems[slot]
        ).wait()

    def fire_wb(bi, slot):
        pltpu.async_copy(
            t_v.at[slot].at[:, :, :, pl.ds(0, 128)],
            out_hbm.at[pl.ds(bi * PB, PB), :, wid],
            osems[slot],
        )

    def wait_wb(slot):
        pltpu.make_async_copy(
            t_v.at[slot].at[:, :, :, pl.ds(0, 128)],
            out_hbm.at[pl.ds(0, PB), :, 0],
            osems[slot],
        ).wait()

    fire_gather(0, 0)

    def outer(g, carry):
        for b in range(2):
            bi = g * 2 + b
            wait_gather(b)

            @pl.when(bi + 1 < NB)
            def _():
                fire_gather(bi + 1, 1 - b)

            @pl.when(bi >= 2)
            def _():
                wait_wb(b)

            rows = rows_v.at[b]
            for j in range(PB):
                tj = t_v.at[b].at[j]

                @plsc.parallel_loop(0, 128, 1, unroll=4)
                def _(l):
                    l_sp = jnp.zeros((16,), jnp.int32) + l
                    for h in range(2):
                        vals = rows[j * 128 + l, pl.ds(h * 16, 16)] * SCALE
                        plsc.store_scatter(
                            tj, [r_idx[h], s_idx, l_sp], vals
                        )

            fire_wb(bi, b)
        return carry

    lax.fori_loop(0, NB // 2, outer, 0)

    for slot in range(2):
        wait_wb(slot)


def kernel(x, table):
    xt = (
        x.astype(jnp.int32)
        .T.reshape(PG, 8, NW, 128)
        .transpose(0, 2, 1, 3)
    )
    tpad = jnp.pad(table, ((0, VPAD - table.shape[0]), (0, 0)))
    tv = tpad.T.reshape(DIM // 8, 8, TC_, 128).transpose(0, 2, 1, 3)
    trm = _detile_sc(tv)
    v = _embed_sc(xt, trm)
    return v.transpose(2, 4, 0, 1, 3).reshape(NA, NP, DIM)
